# baseline (speedup 1.0000x reference)
import functools

import jax
import jax.numpy as jnp
from jax.experimental import pallas as pl


from jax.experimental.pallas import tpu as pltpu


def _gap_kernel(x_ref, o_ref):
    o_ref[0] = jnp.sum(x_ref[0], axis=1)


def _head_kernel(g_ref, w1_ref, b1_ref, w2_ref, b2_ref,
                 ev_ref, ei_ref, loss_ref, *, e, k, eps, scale):
    gap = jnp.sum(g_ref[...], axis=2) * scale
    h = jax.lax.dot_general(
        gap, w1_ref[...], (((1,), (1,)), ((), ())),
        preferred_element_type=jnp.float32)
    h = jax.nn.relu(h + b1_ref[...][None, :])
    pre = jax.lax.dot_general(
        h, w2_ref[...], (((1,), (1,)), ((), ())),
        preferred_element_type=jnp.float32)
    pre = pre + b2_ref[...][None, :]

    m = jnp.max(pre, axis=1, keepdims=True)
    ex = jnp.exp(pre - m)
    logits = ex / jnp.sum(ex, axis=1, keepdims=True)

    b = logits.shape[0]
    ids = jax.lax.broadcasted_iota(jnp.int32, (b, e), 1)

    m1 = jnp.max(logits, axis=1, keepdims=True)
    i1 = jnp.min(jnp.where(logits == m1, ids, e), axis=1, keepdims=True)
    masked = jnp.where(ids == i1, -jnp.inf, logits)
    m2 = jnp.max(masked, axis=1, keepdims=True)
    i2 = jnp.min(jnp.where(masked == m2, ids, e), axis=1, keepdims=True)

    vals = jnp.concatenate([m1, m2], axis=1)
    if k < e:
        ev = jnp.exp(vals - m1)
        vals = ev / jnp.sum(ev, axis=1, keepdims=True)
    ev_ref[...] = vals
    ei_ref[...] = jnp.concatenate([i1, i2], axis=1).astype(jnp.int32)

    n = b * e
    mean = jnp.sum(logits) / n
    var = jnp.sum((logits - mean) ** 2) / (n - 1)
    loss_ref[...] = (jnp.sqrt(var) / (mean + eps)).reshape(1, 1)


def kernel(x, W1, b1, W2, b2):
    B, C, H, W = x.shape
    E = W2.shape[0]
    K = 2
    EPS = 1e-10

    Cb = 32
    while C % Cb:
        Cb //= 2
    nc = C // Cb

    gap3 = pl.pallas_call(
        _gap_kernel,
        grid=(B, nc),
        in_specs=[pl.BlockSpec((1, Cb, H, W), lambda b, c: (b, c, 0, 0))],
        out_specs=pl.BlockSpec((1, Cb, W), lambda b, c: (b, c, 0)),
        out_shape=jax.ShapeDtypeStruct((B, C, W), jnp.float32),
    )(x)

    ev, ei, loss = pl.pallas_call(
        functools.partial(_head_kernel, e=E, k=K, eps=EPS,
                          scale=1.0 / (H * W)),
        out_shape=(
            jax.ShapeDtypeStruct((B, K), jnp.float32),
            jax.ShapeDtypeStruct((B, K), jnp.int32),
            jax.ShapeDtypeStruct((1, 1), jnp.float32),
        ),
    )(gap3, W1, b1, W2, b2)
    return ev, ei, loss[0, 0]


# speedup vs baseline: 4.5395x; 4.5395x over previous
import functools

import jax
import jax.numpy as jnp
from jax.experimental import pallas as pl


def _gap_kernel(x_ref, o_ref):
    o_ref[0, 0, 0] = jnp.sum(x_ref[0], axis=(0, 1))


def _head_kernel(g_ref, w1_ref, b1_ref, w2_ref, b2_ref,
                 ev_ref, ei_ref, loss_ref, *, e, k, eps, scale):
    gap = jnp.sum(g_ref[...], axis=(1, 2)) * scale
    h = jax.lax.dot_general(
        gap, w1_ref[...], (((1,), (1,)), ((), ())),
        preferred_element_type=jnp.float32)
    h = jax.nn.relu(h + b1_ref[...][None, :])
    pre = jax.lax.dot_general(
        h, w2_ref[...], (((1,), (1,)), ((), ())),
        preferred_element_type=jnp.float32)
    pre = pre + b2_ref[...][None, :]

    m = jnp.max(pre, axis=1, keepdims=True)
    ex = jnp.exp(pre - m)
    logits = ex / jnp.sum(ex, axis=1, keepdims=True)

    b = logits.shape[0]
    ids = jax.lax.broadcasted_iota(jnp.int32, (b, e), 1)

    m1 = jnp.max(logits, axis=1, keepdims=True)
    i1 = jnp.min(jnp.where(logits == m1, ids, e), axis=1, keepdims=True)
    masked = jnp.where(ids == i1, -jnp.inf, logits)
    m2 = jnp.max(masked, axis=1, keepdims=True)
    i2 = jnp.min(jnp.where(masked == m2, ids, e), axis=1, keepdims=True)

    vals = jnp.concatenate([m1, m2], axis=1)
    if k < e:
        ev = jnp.exp(vals - m1)
        vals = ev / jnp.sum(ev, axis=1, keepdims=True)
    ev_ref[...] = vals
    ei_ref[...] = jnp.concatenate([i1, i2], axis=1).astype(jnp.int32)

    n = b * e
    mean = jnp.sum(logits) / n
    var = jnp.sum((logits - mean) ** 2) / (n - 1)
    loss_ref[...] = (jnp.sqrt(var) / (mean + eps)).reshape(1, 1)


def kernel(x, W1, b1, W2, b2):
    B, C, H, W = x.shape
    E = W2.shape[0]
    K = 2
    EPS = 1e-10

    xt = jnp.transpose(x, (0, 2, 3, 1))

    Hb = 8 if H % 8 == 0 else H
    nh = H // Hb

    gap3 = pl.pallas_call(
        _gap_kernel,
        grid=(B, nh),
        in_specs=[pl.BlockSpec((1, Hb, W, C), lambda b, h: (b, h, 0, 0))],
        out_specs=pl.BlockSpec((1, 1, 1, C), lambda b, h: (b, h, 0, 0)),
        out_shape=jax.ShapeDtypeStruct((B, nh, 1, C), jnp.float32),
    )(xt)

    ev, ei, loss = pl.pallas_call(
        functools.partial(_head_kernel, e=E, k=K, eps=EPS,
                          scale=1.0 / (H * W)),
        out_shape=(
            jax.ShapeDtypeStruct((B, K), jnp.float32),
            jax.ShapeDtypeStruct((B, K), jnp.int32),
            jax.ShapeDtypeStruct((1, 1), jnp.float32),
        ),
    )(gap3, W1, b1, W2, b2)
    return ev, ei, loss[0, 0]


# speedup vs baseline: 4.6187x; 1.0174x over previous
import functools

import jax
import jax.numpy as jnp
from jax.experimental import pallas as pl


def _gap_kernel(x_ref, o_ref):
    o_ref[0, 0, 0] = jnp.sum(x_ref[0], axis=(0, 1))


def _head_kernel(g_ref, w1_ref, b1_ref, w2_ref, b2_ref,
                 ev_ref, ei_ref, loss_ref, *, e, k, eps, scale):
    gap = jnp.sum(g_ref[...], axis=(1, 2)) * scale
    h = jax.lax.dot_general(
        gap, w1_ref[...], (((1,), (1,)), ((), ())),
        preferred_element_type=jnp.float32)
    h = jax.nn.relu(h + b1_ref[...][None, :])
    pre = jax.lax.dot_general(
        h, w2_ref[...], (((1,), (1,)), ((), ())),
        preferred_element_type=jnp.float32)
    pre = pre + b2_ref[...][None, :]

    m = jnp.max(pre, axis=1, keepdims=True)
    ex = jnp.exp(pre - m)
    logits = ex / jnp.sum(ex, axis=1, keepdims=True)

    b = logits.shape[0]
    ids = jax.lax.broadcasted_iota(jnp.int32, (b, e), 1)

    m1 = jnp.max(logits, axis=1, keepdims=True)
    i1 = jnp.min(jnp.where(logits == m1, ids, e), axis=1, keepdims=True)
    masked = jnp.where(ids == i1, -jnp.inf, logits)
    m2 = jnp.max(masked, axis=1, keepdims=True)
    i2 = jnp.min(jnp.where(masked == m2, ids, e), axis=1, keepdims=True)

    vals = jnp.concatenate([m1, m2], axis=1)
    if k < e:
        ev = jnp.exp(vals - m1)
        vals = ev / jnp.sum(ev, axis=1, keepdims=True)
    ev_ref[...] = vals
    ei_ref[...] = jnp.concatenate([i1, i2], axis=1).astype(jnp.int32)

    n = b * e
    mean = jnp.sum(logits) / n
    var = jnp.sum((logits - mean) ** 2) / (n - 1)
    loss_ref[...] = (jnp.sqrt(var) / (mean + eps)).reshape(1, 1)


def kernel(x, W1, b1, W2, b2):
    B, C, H, W = x.shape
    E = W2.shape[0]
    K = 2
    EPS = 1e-10

    xt = jnp.transpose(x, (0, 2, 3, 1))

    Hb = 16 if H % 16 == 0 else H
    nh = H // Hb

    gap3 = pl.pallas_call(
        _gap_kernel,
        grid=(B, nh),
        in_specs=[pl.BlockSpec((1, Hb, W, C), lambda b, h: (b, h, 0, 0))],
        out_specs=pl.BlockSpec((1, 1, 1, C), lambda b, h: (b, h, 0, 0)),
        out_shape=jax.ShapeDtypeStruct((B, nh, 1, C), jnp.float32),
    )(xt)

    ev, ei, loss = pl.pallas_call(
        functools.partial(_head_kernel, e=E, k=K, eps=EPS,
                          scale=1.0 / (H * W)),
        out_shape=(
            jax.ShapeDtypeStruct((B, K), jnp.float32),
            jax.ShapeDtypeStruct((B, K), jnp.int32),
            jax.ShapeDtypeStruct((1, 1), jnp.float32),
        ),
    )(gap3, W1, b1, W2, b2)
    return ev, ei, loss[0, 0]
